# baseline (speedup 1.0000x reference)
import functools

import jax
import jax.numpy as jnp
from jax import lax
from jax.experimental import pallas as pl
from jax.experimental.pallas import tpu as pltpu
from jax.experimental.pallas import tpu_sc as plsc

_DELTA_V = 0.5
_DELTA_D = 1.5
_ALPHA = 1.0
_BETA = 1.0
_GAMMA = 0.001
_K = 64
_N = 32768
_D = 16

_CHUNK = 2048
_NCHUNK = 4
_ROWS_PER_W = _N // 4
_KT = 4 * _K
_NUM_PAIRS = _K * (_K - 1) / 2.0


def _nsqrt(x):
    i = lax.bitcast_convert_type(x, jnp.int32)
    y = lax.bitcast_convert_type(jnp.int32(0x5F3759DF) - (i >> 1), jnp.float32)
    for _ in range(3):
        y = y * (1.5 - 0.5 * x * y * y)
    return x * y


def _sc_body(emb_hbm, ids_hbm, out_hbm,
             emb_v, ids_v, ones_v, m2d_v, c2d_v,
             sloc_v, cloc_v, mloc_v, part_v,
             sums_sh, cnts_sh, means_sh):
    c = lax.axis_index("c")
    s = lax.axis_index("s")
    bl = s // 4
    part = s % 4
    b = 4 * c + bl
    iota = lax.iota(jnp.int32, 16)
    zero16 = jnp.zeros((16,), jnp.float32)

    def _ones_row(r, _):
        ones_v[r, :] = zero16 + 1.0
        return 0
    lax.fori_loop(0, 128, _ones_row, 0)

    def _zero_row(r, _):
        m2d_v[r, :] = zero16
        return 0
    lax.fori_loop(0, _KT, _zero_row, 0)

    @pl.when(s == 0)
    def _init_tables():
        pltpu.sync_copy(m2d_v, sums_sh)
        pltpu.sync_copy(m2d_v, cnts_sh)

    plsc.subcore_barrier()

    for chunk in range(_NCHUNK):
        row0 = part * _ROWS_PER_W + chunk * _CHUNK
        pltpu.sync_copy(emb_hbm.at[b, pl.ds(row0, _CHUNK), :], emb_v)
        idr0 = part * (_ROWS_PER_W // 128) + chunk * (_CHUNK // 128)
        pltpu.sync_copy(ids_hbm.at[b, pl.ds(idr0, _CHUNK // 128), :],
                        ids_v.at[pl.ds(chunk * (_CHUNK // 128), _CHUNK // 128)])
        for j in range(_CHUNK // 128):
            jg = chunk * (_CHUNK // 128) + j
            pltpu.sync_copy(emb_v.at[pl.ds(j * 128, 128)],
                            sums_sh.at[ids_v.at[jg]], add=True)
            pltpu.sync_copy(ones_v, cnts_sh.at[ids_v.at[jg]], add=True)

    plsc.subcore_barrier()

    pltpu.sync_copy(sums_sh.at[pl.ds(s * 16, 16)], sloc_v)
    pltpu.sync_copy(cnts_sh.at[pl.ds(s * 16, 16)], cloc_v)
    for r in range(16):
        mloc_v[r, :] = sloc_v[r, :] / jnp.maximum(cloc_v[r, :], 1.0)
    pltpu.sync_copy(mloc_v, means_sh.at[pl.ds(s * 16, 16)])

    plsc.subcore_barrier()

    pltpu.sync_copy(means_sh, m2d_v)
    pltpu.sync_copy(cnts_sh, c2d_v)

    dconsts = [jnp.full((16,), d, jnp.int32) for d in range(_D)]

    vacc = zero16
    for chunk in range(_NCHUNK):
        row0 = part * _ROWS_PER_W + chunk * _CHUNK
        pltpu.sync_copy(emb_hbm.at[b, pl.ds(row0, _CHUNK), :], emb_v)

        def _block(t, acc):
            idrow = chunk * (_CHUNK // 128) + (t >> 3)
            idoff = (t & 7) * 16
            ids16 = ids_v[idrow, pl.ds(idoff, 16)]
            riota = iota + t * 16
            d2 = zero16 + 1e-12
            for d in range(_D):
                colv = plsc.load_gather(emb_v, [riota, dconsts[d]])
                mcol = plsc.load_gather(m2d_v, [ids16, dconsts[d]])
                diff = colv - mcol
                d2 = d2 + diff * diff
            cvec = plsc.load_gather(c2d_v, [ids16, dconsts[0]])
            w = 1.0 / jnp.maximum(cvec, 1.0)
            dist = _nsqrt(d2)
            hin = jnp.maximum(dist - _DELTA_V, 0.0)
            return acc + hin * hin * w

        vacc = lax.fori_loop(0, _CHUNK // 16, _block, vacc)
    var_s = jnp.sum(vacc) * (1.0 / _K)

    base_i = part * 16
    krow0 = bl * _K

    def _irow(i, acc):
        i_loc = base_i + i
        mrow = m2d_v[krow0 + i_loc, :]
        mib = [jnp.broadcast_to(mrow[d], (16,)) for d in range(_D)]
        hsum = acc
        for jb in range(4):
            jloc = jb * 16 + iota
            jidx = krow0 + jloc
            sq = jnp.zeros((16,), jnp.float32)
            for d in range(_D):
                mj = plsc.load_gather(m2d_v, [jidx, dconsts[d]])
                dif = mib[d] - mj
                sq = sq + dif * dif
            mask = jloc > i_loc
            pd = _nsqrt(jnp.where(mask, sq, 1.0))
            h = jnp.maximum(2.0 * _DELTA_D - pd, 0.0)
            hsum = hsum + jnp.where(mask, h * h, 0.0)
        return hsum

    dacc = lax.fori_loop(0, 16, _irow, zero16)
    dist_s = jnp.sum(dacc) * (1.0 / _NUM_PAIRS)

    ridx = krow0 + base_i + iota
    r2 = jnp.zeros((16,), jnp.float32) + 1e-12
    for d in range(_D):
        mr = plsc.load_gather(m2d_v, [ridx, dconsts[d]])
        r2 = r2 + mr * mr
    reg_s = jnp.sum(_nsqrt(r2)) * (1.0 / _K)

    row = jnp.where(iota == 0, var_s,
                    jnp.where(iota == 1, dist_s,
                              jnp.where(iota == 2, reg_s, 0.0)))
    part_v[0, :] = row
    pltpu.sync_copy(part_v, out_hbm.at[pl.ds(c * 16 + s, 1)])


@jax.jit
def _sc_call(embeddings, ids_adj):
    mesh = plsc.VectorSubcoreMesh(core_axis_name="c", subcore_axis_name="s")
    f = functools.partial(
        pl.kernel,
        mesh=mesh,
        compiler_params=pltpu.CompilerParams(
            needs_layout_passes=False, use_tc_tiling_on_sc=False),
        out_type=jax.ShapeDtypeStruct((32, 16), jnp.float32),
        scratch_types=[
            pltpu.VMEM((_CHUNK, _D), jnp.float32),
            pltpu.VMEM((64, 128), jnp.int32),
            pltpu.VMEM((128, _D), jnp.float32),
            pltpu.VMEM((_KT, _D), jnp.float32),
            pltpu.VMEM((_KT, _D), jnp.float32),
            pltpu.VMEM((16, _D), jnp.float32),
            pltpu.VMEM((16, _D), jnp.float32),
            pltpu.VMEM((16, _D), jnp.float32),
            pltpu.VMEM((1, _D), jnp.float32),
            pltpu.VMEM_SHARED((_KT, _D), jnp.float32),
            pltpu.VMEM_SHARED((_KT, _D), jnp.float32),
            pltpu.VMEM_SHARED((_KT, _D), jnp.float32),
        ],
    )(_sc_body)
    return f(embeddings, ids_adj)


def kernel(embeddings, instance_ids):
    bsz = embeddings.shape[0]
    off = (jnp.arange(bsz, dtype=jnp.int32) % 4) * _K
    ids_adj = (instance_ids.astype(jnp.int32) + off[:, None]).reshape(
        bsz, _N // 128, 128)
    out = _sc_call(embeddings, ids_adj)
    p = out.reshape(2, 4, 4, 16)
    vb = jnp.sum(p[..., 0], axis=-1).reshape(bsz)
    db = jnp.sum(p[..., 1], axis=-1).reshape(bsz)
    rb = jnp.sum(p[..., 2], axis=-1).reshape(bsz)
    var_loss = jnp.mean(vb)
    dist_loss = jnp.mean(db)
    reg_loss = jnp.mean(rb)
    total = _ALPHA * var_loss + _BETA * dist_loss + _GAMMA * reg_loss
    return (total, var_loss, dist_loss, reg_loss)
